# baseline (speedup 1.0000x reference)
import jax
import jax.numpy as jnp
from jax import lax
from jax.experimental import pallas as pl
from jax.experimental.pallas import tpu as pltpu

_SMOOTH = 100.0
_N = 64 * 512 * 512
_COLS = 1024
_ROWS = _N // _COLS
_BR = 1024
_G = _ROWS // _BR


def _tc_body(yt_ref, yp_ref, oi_ref, os_ref):
    pi = jnp.zeros((8, _COLS), jnp.float32)
    si = jnp.zeros((8, _COLS), jnp.float32)
    for k in range(_BR // 8):
        x = yt_ref[8 * k:8 * k + 8, :]
        y = yp_ref[8 * k:8 * k + 8, :]
        a = jnp.abs(x)
        b = jnp.abs(y)
        m = x != jnp.float32(0.85)
        a = jnp.where(m, a, jnp.float32(0.0))
        b = jnp.where(m, b, jnp.float32(0.0))
        pi = pi + a * b
        si = si + (a + b)
    oi_ref[...] = pi
    os_ref[...] = si


@jax.jit
def _tc_partials(yt, yp):
    return pl.pallas_call(
        _tc_body,
        grid=(_G,),
        in_specs=[
            pl.BlockSpec((_BR, _COLS), lambda i: (i, 0)),
            pl.BlockSpec((_BR, _COLS), lambda i: (i, 0)),
        ],
        out_specs=[
            pl.BlockSpec((8, _COLS), lambda i: (i, 0)),
            pl.BlockSpec((8, _COLS), lambda i: (i, 0)),
        ],
        out_shape=[
            jax.ShapeDtypeStruct((_G * 8, _COLS), jnp.float32),
            jax.ShapeDtypeStruct((_G * 8, _COLS), jnp.float32),
        ],
        compiler_params=pltpu.CompilerParams(
            dimension_semantics=("arbitrary",),
        ),
    )(yt, yp)


def kernel(y_true, y_pred):
    batch_size = y_true.shape[0]
    oi, os = _tc_partials(y_true.reshape(_ROWS, _COLS), y_pred.reshape(_ROWS, _COLS))
    intersection = oi.sum()
    sum_ = os.sum()
    jac = (intersection + _SMOOTH) / (sum_ - intersection + _SMOOTH)
    return (1.0 - jac) * _SMOOTH / batch_size


# speedup vs baseline: 4.1240x; 4.1240x over previous
import jax
import jax.numpy as jnp
from jax import lax
from jax.experimental import pallas as pl
from jax.experimental.pallas import tpu as pltpu

_SMOOTH = 100.0
_N = 64 * 512 * 512
_COLS = 512
_ROWS = _N // _COLS
_BR = 2048
_G = _ROWS // _BR


def _tc_body(yt_ref, yp_ref, oi_ref, os_ref):
    pi = [jnp.zeros((8, 128), jnp.float32) for _ in range(4)]
    si = [jnp.zeros((8, 128), jnp.float32) for _ in range(4)]
    for k in range(_BR // 8):
        x = yt_ref[8 * k:8 * k + 8, :]
        y = yp_ref[8 * k:8 * k + 8, :]
        a = jnp.abs(x)
        b = jnp.abs(y)
        m = x != jnp.float32(0.85)
        a = jnp.where(m, a, jnp.float32(0.0))
        b = jnp.where(m, b, jnp.float32(0.0))
        p = a * b
        s = a + b
        for j in range(4):
            pi[j] = pi[j] + p[:, 128 * j:128 * j + 128]
            si[j] = si[j] + s[:, 128 * j:128 * j + 128]
    oi_ref[...] = jnp.concatenate(pi, axis=1)
    os_ref[...] = jnp.concatenate(si, axis=1)


@jax.jit
def _tc_partials(yt, yp):
    return pl.pallas_call(
        _tc_body,
        grid=(_G,),
        in_specs=[
            pl.BlockSpec((_BR, _COLS), lambda i: (i, 0)),
            pl.BlockSpec((_BR, _COLS), lambda i: (i, 0)),
        ],
        out_specs=[
            pl.BlockSpec((8, _COLS), lambda i: (i, 0)),
            pl.BlockSpec((8, _COLS), lambda i: (i, 0)),
        ],
        out_shape=[
            jax.ShapeDtypeStruct((_G * 8, _COLS), jnp.float32),
            jax.ShapeDtypeStruct((_G * 8, _COLS), jnp.float32),
        ],
        compiler_params=pltpu.CompilerParams(
            dimension_semantics=("arbitrary",),
        ),
    )(yt, yp)


def kernel(y_true, y_pred):
    batch_size = y_true.shape[0]
    oi, os = _tc_partials(y_true.reshape(_ROWS, _COLS), y_pred.reshape(_ROWS, _COLS))
    intersection = oi.sum()
    sum_ = os.sum()
    jac = (intersection + _SMOOTH) / (sum_ - intersection + _SMOOTH)
    return (1.0 - jac) * _SMOOTH / batch_size


# speedup vs baseline: 4.2016x; 1.0188x over previous
import jax
import jax.numpy as jnp
from jax import lax
from jax.experimental import pallas as pl
from jax.experimental.pallas import tpu as pltpu

_SMOOTH = 100.0
_N = 64 * 512 * 512
_COLS = 512
_ROWS = _N // _COLS
_BR = 4096
_G = _ROWS // _BR


def _tc_body(yt_ref, yp_ref, oi_ref, os_ref):
    pi = [jnp.zeros((8, 128), jnp.float32) for _ in range(4)]
    si = [jnp.zeros((8, 128), jnp.float32) for _ in range(4)]
    for k in range(_BR // 8):
        x = yt_ref[8 * k:8 * k + 8, :]
        y = yp_ref[8 * k:8 * k + 8, :]
        a = jnp.abs(x)
        b = jnp.abs(y)
        m = x != jnp.float32(0.85)
        a = jnp.where(m, a, jnp.float32(0.0))
        b = jnp.where(m, b, jnp.float32(0.0))
        p = a * b
        s = a + b
        for j in range(4):
            pi[j] = pi[j] + p[:, 128 * j:128 * j + 128]
            si[j] = si[j] + s[:, 128 * j:128 * j + 128]
    pcat = jnp.concatenate(pi, axis=1)
    scat = jnp.concatenate(si, axis=1)
    i = pl.program_id(0)

    @pl.when(i == 0)
    def _():
        oi_ref[...] = pcat
        os_ref[...] = scat

    @pl.when(i > 0)
    def _():
        oi_ref[...] += pcat
        os_ref[...] += scat


@jax.jit
def _tc_partials(yt, yp):
    return pl.pallas_call(
        _tc_body,
        grid=(_G,),
        in_specs=[
            pl.BlockSpec((_BR, _COLS), lambda i: (i, 0)),
            pl.BlockSpec((_BR, _COLS), lambda i: (i, 0)),
        ],
        out_specs=[
            pl.BlockSpec((8, _COLS), lambda i: (0, 0)),
            pl.BlockSpec((8, _COLS), lambda i: (0, 0)),
        ],
        out_shape=[
            jax.ShapeDtypeStruct((8, _COLS), jnp.float32),
            jax.ShapeDtypeStruct((8, _COLS), jnp.float32),
        ],
        compiler_params=pltpu.CompilerParams(
            dimension_semantics=("arbitrary",),
        ),
    )(yt, yp)


def kernel(y_true, y_pred):
    batch_size = y_true.shape[0]
    oi, os = _tc_partials(y_true.reshape(_ROWS, _COLS), y_pred.reshape(_ROWS, _COLS))
    intersection = oi.sum()
    sum_ = os.sum()
    jac = (intersection + _SMOOTH) / (sum_ - intersection + _SMOOTH)
    return (1.0 - jac) * _SMOOTH / batch_size
